# baseline (speedup 1.0000x reference)
import functools

import jax
import jax.numpy as jnp
from jax import lax
from jax.experimental import pallas as pl
from jax.experimental.pallas import tpu as pltpu
from jax.experimental.pallas import tpu_sc as plsc

V = 1000
D = 1000
DP = 1024
DP2 = DP // 2
B = 4096
NC, NS = 2, 16
NW = NC * NS
BPW = B // NW
C = 16
NCH = BPW // C
LANES = 16
G = 4 * C
NBLK = DP2 // LANES


def _mm_body(a_ref, b_ref, o_ref):
    acc = lax.dot_general(
        a_ref[...], b_ref[...],
        dimension_numbers=(((1,), (1,)), ((), ())),
        preferred_element_type=jnp.float32,
    )
    o_ref[...] = acc.astype(jnp.bfloat16)


def _make_table(codebook, w_perm):
    return pl.pallas_call(
        _mm_body,
        out_shape=jax.ShapeDtypeStruct((V, DP), jnp.bfloat16),
    )(codebook, w_perm)


_sc_mesh = plsc.VectorSubcoreMesh(core_axis_name="c", subcore_axis_name="s")


@functools.partial(
    pl.kernel,
    out_type=jax.ShapeDtypeStruct((B, DP), jnp.float32),
    mesh=_sc_mesh,
    scratch_types=[
        pltpu.VMEM((NCH * G,), jnp.int32),
        pltpu.VMEM((G, DP2), jnp.float32),
        pltpu.VMEM((G, DP2), jnp.float32),
        pltpu.VMEM((C, DP), jnp.float32),
        pltpu.VMEM((C, DP), jnp.float32),
        pltpu.SemaphoreType.DMA,
        pltpu.SemaphoreType.DMA,
        pltpu.SemaphoreType.DMA,
        pltpu.SemaphoreType.DMA,
    ],
)
def _gather_sum(m_hbm, idx_hbm, out_hbm, idx_v, buf_a, buf_b, ob_a, ob_b,
                sg_a, sg_b, st_a, st_b):
    wid = lax.axis_index("s") * NC + lax.axis_index("c")
    base = wid * BPW

    pltpu.sync_copy(idx_hbm.at[pl.ds(wid * (NCH * G), NCH * G)], idx_v)

    bufs = (buf_a, buf_b)
    obufs = (ob_a, ob_b)
    gsems = (sg_a, sg_b)
    ssems = (st_a, st_b)

    def start_gather(j, k):
        return pltpu.async_copy(
            m_hbm.at[idx_v.at[pl.ds(j * G, G)]], bufs[k], gsems[k])

    sh16 = jnp.full((LANES,), 16, jnp.int32)
    msk = jnp.full((LANES,), -65536, jnp.int32)

    def compute(buf, obuf):
        def col(u, carry):
            o = u * LANES
            for r in range(C):
                lo = None
                hi = None
                for q in range(4):
                    w = lax.bitcast_convert_type(
                        buf[q * C + r, pl.ds(o, LANES)], jnp.int32)
                    l = lax.bitcast_convert_type(
                        lax.shift_left(w, sh16), jnp.float32)
                    h = lax.bitcast_convert_type(
                        lax.bitwise_and(w, msk), jnp.float32)
                    lo = l if lo is None else lo + l
                    hi = h if hi is None else hi + h
                obuf[r, pl.ds(2 * o, LANES)] = lo
                obuf[r, pl.ds(2 * o + LANES, LANES)] = hi
            return carry
        lax.fori_loop(0, NBLK, col, 0)

    gh = {0: start_gather(0, 0)}
    sh = {}
    for j in range(NCH):
        k = j % 2
        if j + 1 < NCH:
            gh[j + 1] = start_gather(j + 1, 1 - k)
        gh[j].wait()
        if j - 2 >= 0:
            sh[j - 2].wait()
        compute(bufs[k], obufs[k])
        sh[j] = pltpu.async_copy(
            obufs[k], out_hbm.at[pl.ds(base + j * C, C)], ssems[k])
    sh[NCH - 2].wait()
    sh[NCH - 1].wait()


def _interleave_indices(x1, x2, x4, x5):
    xs = jnp.stack([x1, x2, x4, x5])
    xs = xs.reshape(4, NW, NCH, C)
    xs = jnp.transpose(xs, (1, 2, 0, 3))
    return xs.reshape(B * 4)


def kernel(x1, x2, x4, x5, codebook, W):
    w_pad = jnp.pad(W, ((0, DP - V), (0, 0)))
    w_perm = (w_pad.reshape(NBLK, 2, LANES, V)
              .transpose(0, 2, 1, 3)
              .reshape(DP, V))
    table = _make_table(codebook.astype(jnp.bfloat16),
                        w_perm.astype(jnp.bfloat16))
    packed = lax.bitcast_convert_type(
        table.reshape(V, DP2, 2), jnp.float32)
    idx = _interleave_indices(x1, x2, x4, x5)
    return _gather_sum(packed, idx)[:, :D]


# speedup vs baseline: 1.3457x; 1.3457x over previous
import functools

import jax
import jax.numpy as jnp
from jax import lax
from jax.experimental import pallas as pl
from jax.experimental.pallas import tpu as pltpu
from jax.experimental.pallas import tpu_sc as plsc

V = 1000
D = 1000
DP = 1024
DP2 = DP // 2
B = 4096
NC, NS = 2, 16
NW = NC * NS
BPW = B // NW
C = 16
NCH = BPW // C
LANES = 16
G = 4 * C
NBLK = DP2 // LANES


def _mm_pack_body(a_ref, wlo_ref, whi_ref, o_ref):
    a = a_ref[...].astype(jnp.bfloat16)
    dn = (((1,), (1,)), ((), ()))
    lo = lax.dot_general(a, wlo_ref[...], dn, preferred_element_type=jnp.float32)
    hi = lax.dot_general(a, whi_ref[...], dn, preferred_element_type=jnp.float32)
    lob = lax.bitcast_convert_type(lo, jnp.int32) + jnp.int32(0x8000)
    hib = lax.bitcast_convert_type(hi, jnp.int32) + jnp.int32(0x8000)
    packed = lax.bitwise_or(
        lax.shift_right_logical(lob, 16),
        lax.bitwise_and(hib, jnp.int32(-65536)),
    )
    o_ref[...] = lax.bitcast_convert_type(packed, jnp.float32)


def _make_table(codebook, w_lo, w_hi):
    return pl.pallas_call(
        _mm_pack_body,
        out_shape=jax.ShapeDtypeStruct((V, DP2), jnp.float32),
    )(codebook, w_lo, w_hi)


_sc_mesh = plsc.VectorSubcoreMesh(core_axis_name="c", subcore_axis_name="s")


@functools.partial(
    pl.kernel,
    out_type=jax.ShapeDtypeStruct((B, DP), jnp.float32),
    mesh=_sc_mesh,
    scratch_types=[
        pltpu.VMEM((4 * BPW,), jnp.int32),
        pltpu.VMEM((G, DP2), jnp.float32),
        pltpu.VMEM((G, DP2), jnp.float32),
        pltpu.VMEM((C, DP), jnp.float32),
        pltpu.VMEM((C, DP), jnp.float32),
        pltpu.SemaphoreType.DMA,
        pltpu.SemaphoreType.DMA,
        pltpu.SemaphoreType.DMA,
        pltpu.SemaphoreType.DMA,
    ],
)
def _gather_sum(m_hbm, i1_hbm, i2_hbm, i4_hbm, i5_hbm, out_hbm,
                idx_v, buf_a, buf_b, ob_a, ob_b, sg_a, sg_b, st_a, st_b):
    wid = lax.axis_index("s") * NC + lax.axis_index("c")
    base = wid * BPW

    for q, ih in enumerate((i1_hbm, i2_hbm, i4_hbm, i5_hbm)):
        pltpu.sync_copy(ih.at[pl.ds(base, BPW)], idx_v.at[pl.ds(q * BPW, BPW)])

    bufs = (buf_a, buf_b)
    obufs = (ob_a, ob_b)
    gsems = (sg_a, sg_b)
    ssems = (st_a, st_b)

    def start_gather(j, k):
        return [
            pltpu.async_copy(
                m_hbm.at[idx_v.at[pl.ds(q * BPW + j * C, C)]],
                bufs[k].at[pl.ds(q * C, C)],
                gsems[k],
            )
            for q in range(4)
        ]

    sh16 = jnp.full((LANES,), 16, jnp.int32)
    msk = jnp.full((LANES,), -65536, jnp.int32)

    def compute(buf, obuf):
        def col(u, carry):
            o = u * LANES
            for r in range(C):
                lo = None
                hi = None
                for q in range(4):
                    w = lax.bitcast_convert_type(
                        buf[q * C + r, pl.ds(o, LANES)], jnp.int32)
                    l = lax.bitcast_convert_type(
                        lax.shift_left(w, sh16), jnp.float32)
                    h = lax.bitcast_convert_type(
                        lax.bitwise_and(w, msk), jnp.float32)
                    lo = l if lo is None else lo + l
                    hi = h if hi is None else hi + h
                obuf[r, pl.ds(2 * o, LANES)] = lo
                obuf[r, pl.ds(2 * o + LANES, LANES)] = hi
            return carry
        lax.fori_loop(0, NBLK, col, 0)

    gh = {0: start_gather(0, 0)}
    sh = {}
    for j in range(NCH):
        k = j % 2
        if j + 1 < NCH:
            gh[j + 1] = start_gather(j + 1, 1 - k)
        for h in gh[j]:
            h.wait()
        if j - 2 >= 0:
            sh[j - 2].wait()
        compute(bufs[k], obufs[k])
        sh[j] = pltpu.async_copy(
            obufs[k], out_hbm.at[pl.ds(base + j * C, C)], ssems[k])
    sh[NCH - 2].wait()
    sh[NCH - 1].wait()


def kernel(x1, x2, x4, x5, codebook, W):
    w_pad = jnp.pad(W, ((0, DP - V), (0, 0)))
    wv = w_pad.reshape(NBLK, 2, LANES, V)
    w_lo = wv[:, 0].reshape(DP2, V).astype(jnp.bfloat16)
    w_hi = wv[:, 1].reshape(DP2, V).astype(jnp.bfloat16)
    table = _make_table(codebook, w_lo, w_hi)
    return _gather_sum(table, x1, x2, x4, x5)[:, :D]


# speedup vs baseline: 1.5483x; 1.1506x over previous
import functools

import jax
import jax.numpy as jnp
from jax import lax
from jax.experimental import pallas as pl
from jax.experimental.pallas import tpu as pltpu
from jax.experimental.pallas import tpu_sc as plsc

V = 1000
D = 1000
DP = 1024
DP2 = DP // 2
B = 4096
NC, NS = 2, 16
NW = NC * NS
BPW = B // NW
C = 16
NCH = BPW // C
LANES = 16
G = 4 * C
NBLK = DP2 // LANES


def _mm_pack_body(a_ref, wlo_ref, whi_ref, o_ref):
    a = a_ref[...].astype(jnp.bfloat16)
    dn = (((1,), (1,)), ((), ()))
    lo = lax.dot_general(a, wlo_ref[...], dn, preferred_element_type=jnp.float32)
    hi = lax.dot_general(a, whi_ref[...], dn, preferred_element_type=jnp.float32)
    lob = lax.bitcast_convert_type(lo, jnp.int32) + jnp.int32(0x8000)
    hib = lax.bitcast_convert_type(hi, jnp.int32) + jnp.int32(0x8000)
    packed = lax.bitwise_or(
        lax.shift_right_logical(lob, 16),
        lax.bitwise_and(hib, jnp.int32(-65536)),
    )
    o_ref[...] = lax.bitcast_convert_type(packed, jnp.float32)


def _make_table(codebook, w_lo, w_hi):
    return pl.pallas_call(
        _mm_pack_body,
        out_shape=jax.ShapeDtypeStruct((V, DP2), jnp.float32),
    )(codebook, w_lo, w_hi)


_sc_mesh = plsc.VectorSubcoreMesh(core_axis_name="c", subcore_axis_name="s")


@functools.partial(
    pl.kernel,
    out_type=jax.ShapeDtypeStruct((B, DP), jnp.float32),
    mesh=_sc_mesh,
    scratch_types=[
        pltpu.VMEM((4 * BPW,), jnp.int32),
        pltpu.VMEM((G, DP2), jnp.float32),
        pltpu.VMEM((G, DP2), jnp.float32),
        pltpu.VMEM((C, DP), jnp.float32),
        pltpu.VMEM((C, DP), jnp.float32),
        pltpu.SemaphoreType.DMA,
        pltpu.SemaphoreType.DMA,
        pltpu.SemaphoreType.DMA,
        pltpu.SemaphoreType.DMA,
    ],
)
def _gather_sum(m_hbm, i1_hbm, i2_hbm, i4_hbm, i5_hbm, out_hbm,
                idx_v, buf_a, buf_b, ob_a, ob_b, sg_a, sg_b, st_a, st_b):
    wid = lax.axis_index("s") * NC + lax.axis_index("c")
    base = wid * BPW

    for q, ih in enumerate((i1_hbm, i2_hbm, i4_hbm, i5_hbm)):
        pltpu.sync_copy(ih.at[pl.ds(base, BPW)], idx_v.at[pl.ds(q * BPW, BPW)])

    bufs = (buf_a, buf_b)
    obufs = (ob_a, ob_b)
    gsems = (sg_a, sg_b)
    ssems = (st_a, st_b)

    def start_gather(j, k):
        return [
            pltpu.async_copy(
                m_hbm.at[idx_v.at[pl.ds(q * BPW + j * C, C)]],
                bufs[k].at[pl.ds(q * C, C)],
                gsems[k],
            )
            for q in range(4)
        ]

    sh16 = jnp.full((LANES,), 16, jnp.int32)
    msk = jnp.full((LANES,), -65536, jnp.int32)

    def compute(buf, obuf):
        @plsc.parallel_loop(0, NBLK, unroll=2)
        def col(u):
            o = u * LANES
            for r in range(C):
                lo = None
                hi = None
                for q in range(4):
                    w = lax.bitcast_convert_type(
                        buf[q * C + r, pl.ds(o, LANES)], jnp.int32)
                    l = lax.bitcast_convert_type(
                        lax.shift_left(w, sh16), jnp.float32)
                    h = lax.bitcast_convert_type(
                        lax.bitwise_and(w, msk), jnp.float32)
                    lo = l if lo is None else lo + l
                    hi = h if hi is None else hi + h
                obuf[r, pl.ds(2 * o, LANES)] = lo
                obuf[r, pl.ds(2 * o + LANES, LANES)] = hi

    gh = {0: start_gather(0, 0)}
    sh = {}
    for j in range(NCH):
        k = j % 2
        if j + 1 < NCH:
            gh[j + 1] = start_gather(j + 1, 1 - k)
        for h in gh[j]:
            h.wait()
        if j - 2 >= 0:
            sh[j - 2].wait()
        compute(bufs[k], obufs[k])
        sh[j] = pltpu.async_copy(
            obufs[k], out_hbm.at[pl.ds(base + j * C, C)], ssems[k])
    sh[NCH - 2].wait()
    sh[NCH - 1].wait()


def kernel(x1, x2, x4, x5, codebook, W):
    w_pad = jnp.pad(W, ((0, DP - V), (0, 0)))
    wv = w_pad.reshape(NBLK, 2, LANES, V)
    w_lo = wv[:, 0].reshape(DP2, V).astype(jnp.bfloat16)
    w_hi = wv[:, 1].reshape(DP2, V).astype(jnp.bfloat16)
    table = _make_table(codebook, w_lo, w_hi)
    return _gather_sum(table, x1, x2, x4, x5)[:, :D]


# speedup vs baseline: 1.6773x; 1.0833x over previous
import functools

import jax
import jax.numpy as jnp
from jax import lax
from jax.experimental import pallas as pl
from jax.experimental.pallas import tpu as pltpu
from jax.experimental.pallas import tpu_sc as plsc

V = 1000
D = 1000
DP = 1024
DP2 = DP // 2
B = 4096
NC, NS = 2, 16
NW = NC * NS
BPW = B // NW
C = 16
NCH = BPW // C
LANES = 16
G = 4 * C
NBLK = DP2 // LANES


def _mm_pack_body(a_ref, wlo_ref, whi_ref, o_ref):
    a = a_ref[...].astype(jnp.bfloat16)
    dn = (((1,), (1,)), ((), ()))
    lo = lax.dot_general(a, wlo_ref[...], dn, preferred_element_type=jnp.float32)
    hi = lax.dot_general(a, whi_ref[...], dn, preferred_element_type=jnp.float32)
    lob = lax.bitcast_convert_type(lo, jnp.int32) + jnp.int32(0x8000)
    hib = lax.bitcast_convert_type(hi, jnp.int32) + jnp.int32(0x8000)
    packed = lax.bitwise_or(
        lax.shift_right_logical(lob, 16),
        lax.bitwise_and(hib, jnp.int32(-65536)),
    )
    o_ref[...] = lax.bitcast_convert_type(packed, jnp.float32)


def _make_table(codebook, w_lo, w_hi):
    return pl.pallas_call(
        _mm_pack_body,
        out_shape=jax.ShapeDtypeStruct((V, DP2), jnp.float32),
    )(codebook, w_lo, w_hi)


_sc_mesh = plsc.VectorSubcoreMesh(core_axis_name="c", subcore_axis_name="s")


@functools.partial(
    pl.kernel,
    out_type=jax.ShapeDtypeStruct((B, DP), jnp.float32),
    mesh=_sc_mesh,
    scratch_types=[
        pltpu.VMEM((4 * BPW,), jnp.int32),
        pltpu.VMEM((G, DP2), jnp.float32),
        pltpu.VMEM((G, DP2), jnp.float32),
        pltpu.VMEM((C, DP), jnp.float32),
        pltpu.VMEM((C, DP), jnp.float32),
        pltpu.SemaphoreType.DMA,
        pltpu.SemaphoreType.DMA,
        pltpu.SemaphoreType.DMA,
        pltpu.SemaphoreType.DMA,
    ],
)
def _gather_sum(m_hbm, i1_hbm, i2_hbm, i4_hbm, i5_hbm, out_hbm,
                idx_v, buf_a, buf_b, ob_a, ob_b, sg_a, sg_b, st_a, st_b):
    wid = lax.axis_index("s") * NC + lax.axis_index("c")
    base = wid * BPW

    for q, ih in enumerate((i1_hbm, i2_hbm, i4_hbm, i5_hbm)):
        pltpu.sync_copy(ih.at[pl.ds(base, BPW)], idx_v.at[pl.ds(q * BPW, BPW)])

    bufs = (buf_a, buf_b)
    obufs = (ob_a, ob_b)
    gsems = (sg_a, sg_b)
    ssems = (st_a, st_b)

    def start_gather(j, k):
        return [
            pltpu.async_copy(
                m_hbm.at[idx_v.at[pl.ds(q * BPW + j * C, C)]],
                bufs[k].at[pl.ds(q * C, C)],
                gsems[k],
            )
            for q in range(4)
        ]

    sh16 = jnp.full((LANES,), 16, jnp.int32)

    def compute(buf, obuf):
        @plsc.parallel_loop(0, NBLK, unroll=4)
        def col(u):
            o = u * LANES
            for r in range(C):
                lo = None
                hi = None
                for q in range(4):
                    wf = buf[q * C + r, pl.ds(o, LANES)]
                    w = lax.bitcast_convert_type(wf, jnp.int32)
                    l = lax.bitcast_convert_type(
                        lax.shift_left(w, sh16), jnp.float32)
                    lo = l if lo is None else lo + l
                    hi = wf if hi is None else hi + wf
                obuf[r, pl.ds(2 * o, LANES)] = lo
                obuf[r, pl.ds(2 * o + LANES, LANES)] = hi

    gh = {0: start_gather(0, 0)}
    sh = {}
    for j in range(NCH):
        k = j % 2
        if j + 1 < NCH:
            gh[j + 1] = start_gather(j + 1, 1 - k)
        for h in gh[j]:
            h.wait()
        if j - 2 >= 0:
            sh[j - 2].wait()
        compute(bufs[k], obufs[k])
        sh[j] = pltpu.async_copy(
            obufs[k], out_hbm.at[pl.ds(base + j * C, C)], ssems[k])
    sh[NCH - 2].wait()
    sh[NCH - 1].wait()


def kernel(x1, x2, x4, x5, codebook, W):
    w_pad = jnp.pad(W, ((0, DP - V), (0, 0)))
    wv = w_pad.reshape(NBLK, 2, LANES, V)
    w_lo = wv[:, 0].reshape(DP2, V).astype(jnp.bfloat16)
    w_hi = wv[:, 1].reshape(DP2, V).astype(jnp.bfloat16)
    table = _make_table(codebook, w_lo, w_hi)
    return _gather_sum(table, x1, x2, x4, x5)[:, :D]
